# baseline (speedup 1.0000x reference)
import jax
import jax.numpy as jnp
from jax.experimental import pallas as pl
from jax.experimental.pallas import tpu as pltpu


BLK_M = 1024
NSPLIT = 8
SUB_M = BLK_M // NSPLIT


def _router_block(*refs):
    x_refs = refs[:NSPLIT]
    wt_ref, probs_ref, scores_ref, idx_ref = refs[NSPLIT:]
    wt = wt_ref[...]
    for j in range(NSPLIT):
        logits = jnp.dot(x_refs[j][...], wt, preferred_element_type=jnp.float32)
        m = jnp.max(logits, axis=-1, keepdims=True)
        e = jnp.exp(logits - m)
        s = jnp.sum(e, axis=-1, keepdims=True)
        probs = e / s
        probs_ref[pl.ds(j * SUB_M, SUB_M), :] = probs
        scores_ref[0, 0, pl.ds(j * SUB_M, SUB_M)] = 1.0 / s[:, 0]
        idx_ref[0, 0, pl.ds(j * SUB_M, SUB_M)] = jnp.argmax(probs, axis=-1).astype(
            jnp.int32
        )


@jax.jit
def kernel(x, W):
    n_tokens, d_model = x.shape
    n_gates = W.shape[0]
    grid = (n_tokens // BLK_M,)
    wt = W.T

    def x_spec(j):
        return pl.BlockSpec((SUB_M, d_model), lambda i, j=j: (i * NSPLIT + j, 0))

    probs, scores, idx = pl.pallas_call(
        _router_block,
        grid=grid,
        in_specs=[x_spec(j) for j in range(NSPLIT)]
        + [pl.BlockSpec((d_model, n_gates), lambda i: (0, 0))],
        out_specs=[
            pl.BlockSpec((BLK_M, n_gates), lambda i: (i, 0)),
            pl.BlockSpec((1, 1, BLK_M), lambda i: (i, 0, 0)),
            pl.BlockSpec((1, 1, BLK_M), lambda i: (i, 0, 0)),
        ],
        out_shape=[
            jax.ShapeDtypeStruct((n_tokens, n_gates), jnp.float32),
            jax.ShapeDtypeStruct((n_tokens // BLK_M, 1, BLK_M), jnp.float32),
            jax.ShapeDtypeStruct((n_tokens // BLK_M, 1, BLK_M), jnp.int32),
        ],
        compiler_params=pltpu.CompilerParams(
            dimension_semantics=("parallel",),
        ),
    )(*([x] * NSPLIT + [wt]))
    return idx.reshape(n_tokens), scores.reshape(n_tokens), probs


# speedup vs baseline: 1.0357x; 1.0357x over previous
import jax
import jax.numpy as jnp
from jax.experimental import pallas as pl
from jax.experimental.pallas import tpu as pltpu


BLK_M = 1024
NSPLIT = 8
SUB_M = BLK_M // NSPLIT


def _router_block(*refs):
    x_refs = refs[:NSPLIT]
    wt_ref, probs_ref, scores_ref, idx_ref = refs[NSPLIT:]
    wt = wt_ref[...]
    for j in range(NSPLIT):
        logits = jnp.dot(x_refs[j][...], wt, preferred_element_type=jnp.float32)
        m = jnp.max(logits, axis=-1, keepdims=True)
        e = jnp.exp(logits - m)
        s = jnp.sum(e, axis=-1, keepdims=True)
        probs = e / s
        probs_ref[pl.ds(j * SUB_M, SUB_M), :] = probs
        scores_ref[0, 0, pl.ds(j * SUB_M, SUB_M)] = jnp.max(probs, axis=-1)
        idx_ref[0, 0, pl.ds(j * SUB_M, SUB_M)] = jnp.argmax(probs, axis=-1).astype(
            jnp.int32
        )


@jax.jit
def kernel(x, W):
    n_tokens, d_model = x.shape
    n_gates = W.shape[0]
    grid = (n_tokens // BLK_M,)
    wt = W.T

    def x_spec(j):
        return pl.BlockSpec((SUB_M, d_model), lambda i, j=j: (i * NSPLIT + j, 0))

    probs, scores, idx = pl.pallas_call(
        _router_block,
        grid=grid,
        in_specs=[x_spec(j) for j in range(NSPLIT)]
        + [pl.BlockSpec((d_model, n_gates), lambda i: (0, 0))],
        out_specs=[
            pl.BlockSpec((BLK_M, n_gates), lambda i: (i, 0)),
            pl.BlockSpec((1, 1, BLK_M), lambda i: (i, 0, 0)),
            pl.BlockSpec((1, 1, BLK_M), lambda i: (i, 0, 0)),
        ],
        out_shape=[
            jax.ShapeDtypeStruct((n_tokens, n_gates), jnp.float32),
            jax.ShapeDtypeStruct((n_tokens // BLK_M, 1, BLK_M), jnp.float32),
            jax.ShapeDtypeStruct((n_tokens // BLK_M, 1, BLK_M), jnp.int32),
        ],
        compiler_params=pltpu.CompilerParams(
            dimension_semantics=("parallel",),
        ),
    )(*([x] * NSPLIT + [wt]))
    return idx.reshape(n_tokens), scores.reshape(n_tokens), probs


# speedup vs baseline: 1.0437x; 1.0077x over previous
import jax
import jax.numpy as jnp
from jax.experimental import pallas as pl
from jax.experimental.pallas import tpu as pltpu


BLK_M = 1024
NSPLIT = 4
SUB_M = BLK_M // NSPLIT


def _router_block(*refs):
    x_refs = refs[:NSPLIT]
    wt_ref, probs_ref, scores_ref, idx_ref = refs[NSPLIT:]
    wt = wt_ref[...]
    for j in range(NSPLIT):
        logits = jnp.dot(x_refs[j][...], wt, preferred_element_type=jnp.float32)
        m = jnp.max(logits, axis=-1, keepdims=True)
        e = jnp.exp(logits - m)
        s = jnp.sum(e, axis=-1, keepdims=True)
        probs = e / s
        probs_ref[pl.ds(j * SUB_M, SUB_M), :] = probs
        scores_ref[0, 0, pl.ds(j * SUB_M, SUB_M)] = jnp.max(probs, axis=-1)
        idx_ref[0, 0, pl.ds(j * SUB_M, SUB_M)] = jnp.argmax(probs, axis=-1).astype(
            jnp.int32
        )


@jax.jit
def kernel(x, W):
    n_tokens, d_model = x.shape
    n_gates = W.shape[0]
    grid = (n_tokens // BLK_M,)
    wt = W.T

    def x_spec(j):
        return pl.BlockSpec((SUB_M, d_model), lambda i, j=j: (i * NSPLIT + j, 0))

    probs, scores, idx = pl.pallas_call(
        _router_block,
        grid=grid,
        in_specs=[x_spec(j) for j in range(NSPLIT)]
        + [pl.BlockSpec((d_model, n_gates), lambda i: (0, 0))],
        out_specs=[
            pl.BlockSpec((BLK_M, n_gates), lambda i: (i, 0)),
            pl.BlockSpec((1, 1, BLK_M), lambda i: (i, 0, 0)),
            pl.BlockSpec((1, 1, BLK_M), lambda i: (i, 0, 0)),
        ],
        out_shape=[
            jax.ShapeDtypeStruct((n_tokens, n_gates), jnp.float32),
            jax.ShapeDtypeStruct((n_tokens // BLK_M, 1, BLK_M), jnp.float32),
            jax.ShapeDtypeStruct((n_tokens // BLK_M, 1, BLK_M), jnp.int32),
        ],
        compiler_params=pltpu.CompilerParams(
            dimension_semantics=("parallel",),
        ),
    )(*([x] * NSPLIT + [wt]))
    return idx.reshape(n_tokens), scores.reshape(n_tokens), probs
